# baseline (speedup 1.0000x reference)
import jax
import jax.numpy as jnp
from jax.experimental import pallas as pl
from jax.experimental.pallas import tpu as pltpu

N = 20000
MAX_OUT = 300
IOU_THRESH = 0.5
LANES = 128
ROWS = 160
NPAD = ROWS * LANES
NEG = -1e30
T = 4
CH = 32
NCH = ROWS // CH


def _comb(am, ai, bm, bi):
    m = jnp.maximum(am, bm)
    i = jnp.where(am > bm, ai, jnp.where(bm > am, bi, jnp.minimum(ai, bi)))
    return m, i


def _nms_body(x1_ref, y1_ref, x2_ref, y2_ref, sc_ref,
              ob_ref, os_ref, oi_ref, live_ref, area_ref):
    live_ref[...] = sc_ref[...]
    area_ref[...] = (x2_ref[...] - x1_ref[...]) * (y2_ref[...] - y1_ref[...])
    ob_ref[...] = jnp.zeros_like(ob_ref)
    os_ref[...] = jnp.zeros_like(os_ref)
    oi_ref[...] = jnp.full_like(oi_ref, -1)

    r32 = jax.lax.broadcasted_iota(jnp.int32, (CH, LANES), 0)
    c32 = jax.lax.broadcasted_iota(jnp.int32, (CH, LANES), 1)
    idx_t = r32 * LANES + c32
    lane_iota = jax.lax.broadcasted_iota(jnp.int32, (1, LANES), 1)
    four_iota = jax.lax.broadcasted_iota(jnp.int32, (1, 4), 1)

    def argmax_pass(ps_prev):
        am = jnp.full((CH, LANES), NEG, jnp.float32)
        ai = jnp.zeros((CH, LANES), jnp.int32)
        for b in range(NCH):
            v = live_ref[b * CH:(b + 1) * CH, :]
            idx = idx_t + b * CH * LANES
            for p in ps_prev:
                v = jnp.where(idx == p, NEG, v)
            gt = v > am
            ai = jnp.where(gt, idx, ai)
            am = jnp.maximum(am, v)
        am, ai = _comb(am[0:16], ai[0:16], am[16:32], ai[16:32])
        am, ai = _comb(am[0:8], ai[0:8], am[8:16], ai[8:16])
        for axis, shifts in ((0, (1, 2, 4)), (1, (1, 2, 4, 8, 16, 32, 64))):
            for s in shifts:
                bm = pltpu.roll(am, s, axis)
                bi = pltpu.roll(ai, s, axis)
                am, ai = _comb(am, ai, bm, bi)
        return am[0:1, 0:1], ai[0:1, 0:1]

    def round_body(carry):
        k, rnd = carry

        ms, ps = [], []
        for j in range(T):
            m_j, p_j = argmax_pass(ps)
            ms.append(m_j)
            ps.append(p_j)

        pk_a = ps[0] + ps[1] * 32768
        pk_b = ps[2] + ps[3] * 32768
        sa = pk_a[0, 0]
        sb = pk_b[0, 0]

        bxs = []
        for j, (word, lo) in enumerate(((sa, True), (sa, False),
                                        (sb, True), (sb, False))):
            pjs = word & 32767 if lo else word >> 15
            r = pjs >> 7
            c = pjs & 127
            lmask = lane_iota == c

            def pick(ref, r=r, lmask=lmask):
                return jnp.sum(jnp.where(lmask, ref[pl.ds(r, 1), :], 0.0),
                               axis=1, keepdims=True)

            bx1 = pick(x1_ref)
            by1 = pick(y1_ref)
            bx2 = pick(x2_ref)
            by2 = pick(y2_ref)
            bxs.append((bx1, by1, bx2, by2, (bx2 - bx1) * (by2 - by1)))

        def pair_iou(i, j):
            ax1, ay1, ax2, ay2, aa = bxs[i]
            bx1, by1, bx2, by2, ba = bxs[j]
            iw = jnp.maximum(jnp.minimum(ax2, bx2) - jnp.maximum(ax1, bx1), 0.0)
            ih = jnp.maximum(jnp.minimum(ay2, by2) - jnp.maximum(ay1, by1), 0.0)
            inter = iw * ih
            return inter / (aa + ba - inter + 1e-6)

        committed = [ms[0] > -0.5]
        for j in range(1, T):
            ok = ms[j] > -0.5
            for i in range(j):
                ok = ok & (jnp.logical_not(committed[i])
                           | jnp.logical_not(pair_iou(i, j) > IOU_THRESH))
            committed.append(ok)

        pk = committed[0].astype(jnp.int32)
        for j in range(1, T):
            pk = pk + committed[j].astype(jnp.int32) * (1 << j)
        pks = pk[0, 0]

        for b in range(NCH):
            sl = pl.ds(b * CH, CH)
            x1c = x1_ref[sl, :]
            y1c = y1_ref[sl, :]
            x2c = x2_ref[sl, :]
            y2c = y2_ref[sl, :]
            areac = area_ref[sl, :]
            lv = live_ref[sl, :]
            idx = idx_t + b * CH * LANES
            for j in range(T):
                bx1, by1, bx2, by2, ba = bxs[j]
                iw = jnp.maximum(jnp.minimum(bx2, x2c) - jnp.maximum(bx1, x1c), 0.0)
                ih = jnp.maximum(jnp.minimum(by2, y2c) - jnp.maximum(by1, y1c), 0.0)
                inter = iw * ih
                iou = inter / (ba + areac - inter + 1e-6)
                kill = ((iou > IOU_THRESH) & committed[j]) | (idx == ps[j])
                lv = jnp.where(kill, NEG, lv)
            live_ref[sl, :] = lv

        slot = k
        for j in range(T):
            cbit = (pks >> j) & 1

            @pl.when((cbit == 1) & (slot < MAX_OUT))
            def _():
                sclamp = jnp.minimum(slot, MAX_OUT - 1)
                os_ref[pl.ds(sclamp, 1), :] = ms[j]
                oi_ref[pl.ds(sclamp, 1), :] = ps[j]
                brow = jnp.where(four_iota == 0, bxs[j][0],
                       jnp.where(four_iota == 1, bxs[j][1],
                       jnp.where(four_iota == 2, bxs[j][2], bxs[j][3])))
                ob_ref[pl.ds(sclamp, 1), :] = brow

            slot = slot + cbit

        return slot, rnd + 1

    def cond(carry):
        k, rnd = carry
        return (k < MAX_OUT) & (rnd < MAX_OUT)

    jax.lax.while_loop(cond, round_body, (jnp.int32(0), jnp.int32(0)))


def _run(x1, y1, x2, y2, sc, interpret=False):
    return pl.pallas_call(
        _nms_body,
        out_shape=(
            jax.ShapeDtypeStruct((MAX_OUT, 4), jnp.float32),
            jax.ShapeDtypeStruct((MAX_OUT, 1), jnp.float32),
            jax.ShapeDtypeStruct((MAX_OUT, 1), jnp.int32),
        ),
        scratch_shapes=[pltpu.VMEM((ROWS, LANES), jnp.float32),
                        pltpu.VMEM((ROWS, LANES), jnp.float32)],
        interpret=interpret,
    )(x1, y1, x2, y2, sc)


@jax.jit
def kernel(boxes, scores):
    pad = NPAD - N
    x1 = jnp.pad(boxes[:, 0], (0, pad)).reshape(ROWS, LANES)
    y1 = jnp.pad(boxes[:, 1], (0, pad)).reshape(ROWS, LANES)
    x2 = jnp.pad(boxes[:, 2], (0, pad)).reshape(ROWS, LANES)
    y2 = jnp.pad(boxes[:, 3], (0, pad)).reshape(ROWS, LANES)
    sc = jnp.pad(scores, (0, pad), constant_values=NEG).reshape(ROWS, LANES)
    ob, os_, oi = _run(x1, y1, x2, y2, sc)
    return ob, os_.reshape(MAX_OUT), oi.reshape(MAX_OUT)


# speedup vs baseline: 1.8930x; 1.8930x over previous
import jax
import jax.numpy as jnp
from jax.experimental import pallas as pl
from jax.experimental.pallas import tpu as pltpu

N = 20000
MAX_OUT = 300
IOU_THRESH = 0.5
LANES = 128
ROWS = 160
NPAD = ROWS * LANES
NEG = -1e30
T = 4


def _tree(a, op2, red):
    a = op2(a[0:80], a[80:160])
    a = op2(a[0:40], a[40:80])
    a = red(a, axis=0, keepdims=True)
    return red(a, axis=1, keepdims=True)


def _tsum(a):
    return _tree(a, jnp.add, jnp.sum)


def _nms_body(x1_ref, y1_ref, x2_ref, y2_ref, sc_ref,
              ob_ref, os_ref, oi_ref, live_ref, area_ref):
    live_ref[...] = sc_ref[...]
    area_ref[...] = (x2_ref[...] - x1_ref[...]) * (y2_ref[...] - y1_ref[...])
    ob_ref[...] = jnp.zeros_like(ob_ref)
    os_ref[...] = jnp.zeros_like(os_ref)
    oi_ref[...] = jnp.full_like(oi_ref, -1)

    idx2d = jax.lax.broadcasted_iota(jnp.int32, (ROWS, LANES), 0) * LANES \
        + jax.lax.broadcasted_iota(jnp.int32, (ROWS, LANES), 1)
    four_iota = jax.lax.broadcasted_iota(jnp.int32, (1, 4), 1)

    def argmax_stage():
        v = live_ref[...]
        colmax = jnp.max(v, axis=0, keepdims=True)
        pidx_col = jnp.min(jnp.where(v == colmax, idx2d, NPAD),
                           axis=0, keepdims=True)
        m = jnp.max(colmax, axis=1, keepdims=True)
        p = jnp.min(jnp.where(colmax == m, pidx_col, NPAD),
                    axis=1, keepdims=True)
        live_ref[...] = jnp.where(idx2d == p, NEG, v)

        rowmask = idx2d == pidx_col
        cvals = jnp.concatenate(
            [jnp.sum(jnp.where(rowmask, ref[...], 0.0), axis=0, keepdims=True)
             for ref in (x1_ref, y1_ref, x2_ref, y2_ref)], axis=0)
        vals = jnp.sum(jnp.where(pidx_col == p, cvals, 0.0),
                       axis=1, keepdims=True)
        return m, p, vals

    def round_body(carry):
        k, rnd = carry

        ms, ps, bxs = [], [], []
        for j in range(T):
            m_j, p_j, vals = argmax_stage()
            bx1 = vals[0:1, 0:1]
            by1 = vals[1:2, 0:1]
            bx2 = vals[2:3, 0:1]
            by2 = vals[3:4, 0:1]
            ms.append(m_j)
            ps.append(p_j)
            bxs.append((bx1, by1, bx2, by2, (bx2 - bx1) * (by2 - by1)))

        def pair_iou(i, j):
            ax1, ay1, ax2, ay2, aa = bxs[i]
            bx1, by1, bx2, by2, ba = bxs[j]
            iw = jnp.maximum(jnp.minimum(ax2, bx2) - jnp.maximum(ax1, bx1), 0.0)
            ih = jnp.maximum(jnp.minimum(ay2, by2) - jnp.maximum(ay1, by1), 0.0)
            inter = iw * ih
            return inter / (aa + ba - inter + 1e-6)

        committed = [ms[0] > -0.5]
        for j in range(1, T):
            ok = ms[j] > -0.5
            for i in range(j):
                ok = ok & (jnp.logical_not(committed[i])
                           | jnp.logical_not(pair_iou(i, j) > IOU_THRESH))
            committed.append(ok)

        pk = committed[0].astype(jnp.int32)
        for j in range(1, T):
            pk = pk + committed[j].astype(jnp.int32) * (1 << j)
        pks = pk[0, 0]

        area = area_ref[...]
        x1 = x1_ref[...]
        y1 = y1_ref[...]
        x2 = x2_ref[...]
        y2 = y2_ref[...]
        newlive = live_ref[...]
        for j in range(T):
            bx1, by1, bx2, by2, ba = bxs[j]
            iw = jnp.maximum(jnp.minimum(bx2, x2) - jnp.maximum(bx1, x1), 0.0)
            ih = jnp.maximum(jnp.minimum(by2, y2) - jnp.maximum(by1, y1), 0.0)
            inter = iw * ih
            iou = inter / (ba + area - inter + 1e-6)
            newlive = jnp.where((iou > IOU_THRESH) & committed[j], NEG, newlive)
        live_ref[...] = newlive

        slot = k
        for j in range(T):
            cbit = (pks >> j) & 1

            @pl.when((cbit == 1) & (slot < MAX_OUT))
            def _():
                sclamp = jnp.minimum(slot, MAX_OUT - 1)
                os_ref[pl.ds(sclamp, 1), :] = ms[j]
                oi_ref[pl.ds(sclamp, 1), :] = ps[j]
                brow = jnp.where(four_iota == 0, bxs[j][0],
                       jnp.where(four_iota == 1, bxs[j][1],
                       jnp.where(four_iota == 2, bxs[j][2], bxs[j][3])))
                ob_ref[pl.ds(sclamp, 1), :] = brow

            slot = slot + cbit

        return slot, rnd + 1

    def cond(carry):
        k, rnd = carry
        return (k < MAX_OUT) & (rnd < MAX_OUT)

    jax.lax.while_loop(cond, round_body, (jnp.int32(0), jnp.int32(0)))


def _run(x1, y1, x2, y2, sc, interpret=False):
    return pl.pallas_call(
        _nms_body,
        out_shape=(
            jax.ShapeDtypeStruct((MAX_OUT, 4), jnp.float32),
            jax.ShapeDtypeStruct((MAX_OUT, 1), jnp.float32),
            jax.ShapeDtypeStruct((MAX_OUT, 1), jnp.int32),
        ),
        scratch_shapes=[pltpu.VMEM((ROWS, LANES), jnp.float32),
                        pltpu.VMEM((ROWS, LANES), jnp.float32)],
        interpret=interpret,
    )(x1, y1, x2, y2, sc)


@jax.jit
def kernel(boxes, scores):
    pad = NPAD - N
    x1 = jnp.pad(boxes[:, 0], (0, pad)).reshape(ROWS, LANES)
    y1 = jnp.pad(boxes[:, 1], (0, pad)).reshape(ROWS, LANES)
    x2 = jnp.pad(boxes[:, 2], (0, pad)).reshape(ROWS, LANES)
    y2 = jnp.pad(boxes[:, 3], (0, pad)).reshape(ROWS, LANES)
    sc = jnp.pad(scores, (0, pad), constant_values=NEG).reshape(ROWS, LANES)
    ob, os_, oi = _run(x1, y1, x2, y2, sc)
    return ob, os_.reshape(MAX_OUT), oi.reshape(MAX_OUT)


# speedup vs baseline: 1.9482x; 1.0292x over previous
import jax
import jax.numpy as jnp
from jax.experimental import pallas as pl
from jax.experimental.pallas import tpu as pltpu

N = 20000
MAX_OUT = 300
IOU_THRESH = 0.5
LANES = 128
ROWS = 160
NPAD = ROWS * LANES
NEG = -1e30
T = 6


def _tree(a, op2, red):
    a = op2(a[0:80], a[80:160])
    a = op2(a[0:40], a[40:80])
    a = red(a, axis=0, keepdims=True)
    return red(a, axis=1, keepdims=True)


def _tsum(a):
    return _tree(a, jnp.add, jnp.sum)


def _nms_body(x1_ref, y1_ref, x2_ref, y2_ref, sc_ref,
              ob_ref, os_ref, oi_ref, live_ref, area_ref):
    live_ref[...] = sc_ref[...]
    area_ref[...] = (x2_ref[...] - x1_ref[...]) * (y2_ref[...] - y1_ref[...])
    ob_ref[...] = jnp.zeros_like(ob_ref)
    os_ref[...] = jnp.zeros_like(os_ref)
    oi_ref[...] = jnp.full_like(oi_ref, -1)

    idx2d = jax.lax.broadcasted_iota(jnp.int32, (ROWS, LANES), 0) * LANES \
        + jax.lax.broadcasted_iota(jnp.int32, (ROWS, LANES), 1)
    four_iota = jax.lax.broadcasted_iota(jnp.int32, (1, 4), 1)

    def argmax_stage():
        v = live_ref[...]
        colmax = jnp.max(v, axis=0, keepdims=True)
        pidx_col = jnp.min(jnp.where(v == colmax, idx2d, NPAD),
                           axis=0, keepdims=True)
        m = jnp.max(colmax, axis=1, keepdims=True)
        p = jnp.min(jnp.where(colmax == m, pidx_col, NPAD),
                    axis=1, keepdims=True)
        live_ref[...] = jnp.where(idx2d == p, NEG, v)

        rowmask = idx2d == pidx_col
        cvals = jnp.concatenate(
            [jnp.sum(jnp.where(rowmask, ref[...], 0.0), axis=0, keepdims=True)
             for ref in (x1_ref, y1_ref, x2_ref, y2_ref)], axis=0)
        vals = jnp.sum(jnp.where(pidx_col == p, cvals, 0.0),
                       axis=1, keepdims=True)
        return m, p, vals

    def round_body(carry):
        k, rnd = carry

        ms, ps, bxs = [], [], []
        for j in range(T):
            m_j, p_j, vals = argmax_stage()
            bx1 = vals[0:1, 0:1]
            by1 = vals[1:2, 0:1]
            bx2 = vals[2:3, 0:1]
            by2 = vals[3:4, 0:1]
            ms.append(m_j)
            ps.append(p_j)
            bxs.append((bx1, by1, bx2, by2, (bx2 - bx1) * (by2 - by1)))

        def pair_iou(i, j):
            ax1, ay1, ax2, ay2, aa = bxs[i]
            bx1, by1, bx2, by2, ba = bxs[j]
            iw = jnp.maximum(jnp.minimum(ax2, bx2) - jnp.maximum(ax1, bx1), 0.0)
            ih = jnp.maximum(jnp.minimum(ay2, by2) - jnp.maximum(ay1, by1), 0.0)
            inter = iw * ih
            return inter / (aa + ba - inter + 1e-6)

        committed = [ms[0] > -0.5]
        for j in range(1, T):
            ok = ms[j] > -0.5
            for i in range(j):
                ok = ok & (jnp.logical_not(committed[i])
                           | jnp.logical_not(pair_iou(i, j) > IOU_THRESH))
            committed.append(ok)

        pk = committed[0].astype(jnp.int32)
        for j in range(1, T):
            pk = pk + committed[j].astype(jnp.int32) * (1 << j)
        pks = pk[0, 0]

        area = area_ref[...]
        x1 = x1_ref[...]
        y1 = y1_ref[...]
        x2 = x2_ref[...]
        y2 = y2_ref[...]
        newlive = live_ref[...]
        for j in range(T):
            bx1, by1, bx2, by2, ba = bxs[j]
            iw = jnp.maximum(jnp.minimum(bx2, x2) - jnp.maximum(bx1, x1), 0.0)
            ih = jnp.maximum(jnp.minimum(by2, y2) - jnp.maximum(by1, y1), 0.0)
            inter = iw * ih
            iou = inter / (ba + area - inter + 1e-6)
            newlive = jnp.where((iou > IOU_THRESH) & committed[j], NEG, newlive)
        live_ref[...] = newlive

        slot = k
        for j in range(T):
            cbit = (pks >> j) & 1

            @pl.when((cbit == 1) & (slot < MAX_OUT))
            def _():
                sclamp = jnp.minimum(slot, MAX_OUT - 1)
                os_ref[pl.ds(sclamp, 1), :] = ms[j]
                oi_ref[pl.ds(sclamp, 1), :] = ps[j]
                brow = jnp.where(four_iota == 0, bxs[j][0],
                       jnp.where(four_iota == 1, bxs[j][1],
                       jnp.where(four_iota == 2, bxs[j][2], bxs[j][3])))
                ob_ref[pl.ds(sclamp, 1), :] = brow

            slot = slot + cbit

        return slot, rnd + 1

    def cond(carry):
        k, rnd = carry
        return (k < MAX_OUT) & (rnd < MAX_OUT)

    jax.lax.while_loop(cond, round_body, (jnp.int32(0), jnp.int32(0)))


def _run(x1, y1, x2, y2, sc, interpret=False):
    return pl.pallas_call(
        _nms_body,
        out_shape=(
            jax.ShapeDtypeStruct((MAX_OUT, 4), jnp.float32),
            jax.ShapeDtypeStruct((MAX_OUT, 1), jnp.float32),
            jax.ShapeDtypeStruct((MAX_OUT, 1), jnp.int32),
        ),
        scratch_shapes=[pltpu.VMEM((ROWS, LANES), jnp.float32),
                        pltpu.VMEM((ROWS, LANES), jnp.float32)],
        interpret=interpret,
    )(x1, y1, x2, y2, sc)


@jax.jit
def kernel(boxes, scores):
    pad = NPAD - N
    x1 = jnp.pad(boxes[:, 0], (0, pad)).reshape(ROWS, LANES)
    y1 = jnp.pad(boxes[:, 1], (0, pad)).reshape(ROWS, LANES)
    x2 = jnp.pad(boxes[:, 2], (0, pad)).reshape(ROWS, LANES)
    y2 = jnp.pad(boxes[:, 3], (0, pad)).reshape(ROWS, LANES)
    sc = jnp.pad(scores, (0, pad), constant_values=NEG).reshape(ROWS, LANES)
    ob, os_, oi = _run(x1, y1, x2, y2, sc)
    return ob, os_.reshape(MAX_OUT), oi.reshape(MAX_OUT)
